# baseline (speedup 1.0000x reference)
import jax
import jax.numpy as jnp
from jax import lax
from jax.experimental import pallas as pl
from jax.experimental.pallas import tpu as pltpu
from jax.experimental.pallas import tpu_sc as plsc

A = 26
AH = A // 2
V = 100000
D = 64
B = 16384
TOTAL = B * A
HTOT = B * AH

NC, NS = 2, 16
NW = NC * NS

IDXW = 128
CHUNK = 512
NJ = CHUNK // IDXW
NHALF = HTOT // NW // CHUNK
HCHUNKS = HTOT // CHUNK


def _gather_body(idx_hbm, didx_hbm, tab1_hbm, tab2_hbm, out_hbm,
                 idx_v, didx_v, rows_v, gsem, wsem):
    wid = lax.axis_index("s") * NC + lax.axis_index("c")

    def make_pipeline(tab_hbm, cb):
        def stage(c, s):
            base = cb + wid * NHALF + c
            pltpu.sync_copy(idx_hbm.at[base], idx_v.at[s])
            pltpu.sync_copy(didx_hbm.at[base], didx_v.at[s])
            for j in range(NJ):
                pltpu.async_copy(
                    tab_hbm.at[idx_v.at[s, pl.ds(j * IDXW, IDXW)]],
                    rows_v.at[s, pl.ds(j * IDXW, IDXW)],
                    gsem,
                )

        def wait_gathers(s):
            for j in range(NJ):
                pltpu.make_async_copy(
                    tab_hbm.at[idx_v.at[s, pl.ds(j * IDXW, IDXW)]],
                    rows_v.at[s, pl.ds(j * IDXW, IDXW)],
                    gsem,
                ).wait()

        def fire_writes(s):
            for j in range(NJ):
                pltpu.async_copy(
                    rows_v.at[s, pl.ds(j * IDXW, IDXW)],
                    out_hbm.at[didx_v.at[s, j]],
                    wsem,
                )

        def wait_writes(s):
            for j in range(NJ):
                pltpu.make_async_copy(
                    rows_v.at[s, pl.ds(j * IDXW, IDXW)],
                    out_hbm.at[didx_v.at[s, j]],
                    wsem,
                ).wait()

        def run():
            stage(0, 0)

            def step(i, _):
                for s in range(2):
                    c = 2 * i + s

                    @pl.when(c + 1 < NHALF)
                    def _():
                        @pl.when(c >= 1)
                        def _():
                            wait_writes(1 - s)

                        stage(c + 1, 1 - s)

                    @pl.when(c < NHALF)
                    def _():
                        wait_gathers(s)
                        fire_writes(s)
                return ()

            lax.fori_loop(0, (NHALF + 1) // 2, step, ())
            wait_writes((NHALF - 2) % 2)
            wait_writes((NHALF - 1) % 2)

        return run

    make_pipeline(tab1_hbm, 0)()
    make_pipeline(tab2_hbm, HCHUNKS)()


def _gather(flat_idx, dst_idx, unit_tab1, unit_tab2):
    mesh = plsc.VectorSubcoreMesh(core_axis_name="c", subcore_axis_name="s")
    f = pl.kernel(
        _gather_body,
        out_type=jax.ShapeDtypeStruct((TOTAL, D), jnp.float32),
        mesh=mesh,
        scratch_types=[
            pltpu.VMEM((2, CHUNK), jnp.int32),
            pltpu.VMEM((2, NJ, IDXW), jnp.int32),
            pltpu.VMEM((2, CHUNK, D), jnp.float32),
            pltpu.SemaphoreType.DMA,
            pltpu.SemaphoreType.DMA,
        ],
        compiler_params=pltpu.CompilerParams(
            use_tc_tiling_on_sc=False, needs_layout_passes=False
        ),
    )
    return f(flat_idx, dst_idx, unit_tab1, unit_tab2)


def _half_indices(tuples_h, a0):
    offs = (jnp.arange(AH, dtype=jnp.int32) * (2 * V))[None, :]
    gidx = (2 * tuples_h + offs).reshape(HCHUNKS, CHUNK)
    r = jnp.arange(HTOT, dtype=jnp.int32)
    b, al = r // AH, r % AH
    a = al + a0
    dst = (b >> 3) * (16 * (A // 2)) + (a >> 1) * 16 + (b & 7) * 2 + (a & 1)
    return gidx, dst.reshape(HCHUNKS, NJ, IDXW)


def kernel(tuples, tables):
    u1 = jnp.pad(tables[:AH], ((0, 0), (0, 0), (0, D))).reshape(2 * AH * V, D)
    u2 = jnp.pad(tables[AH:], ((0, 0), (0, 0), (0, D))).reshape(2 * AH * V, D)
    g1, d1 = _half_indices(tuples[:, :AH], 0)
    g2, d2 = _half_indices(tuples[:, AH:], AH)
    flat_idx = jnp.concatenate([g1, g2], axis=0)
    dst_idx = jnp.concatenate([d1, d2], axis=0)
    out = _gather(flat_idx, dst_idx, u1, u2)
    y = out.reshape(B // 8, A // 2, 8, 2 * D)
    return y.transpose(0, 2, 1, 3).reshape(B, A * D)
